# baseline (speedup 1.0000x reference)
import functools

import jax
import jax.numpy as jnp
from jax import lax
from jax.experimental import pallas as pl
from jax.experimental.pallas import tpu as pltpu
from jax.experimental.pallas import tpu_sc as plsc

N = 10000
E = 320000
D = 128
G = 64

NC = 2
NS = 16
NW = NC * NS
CH = 128
CPW = 80
PCH = 40
E_PAD = CH * CPW * NW
N_PAD = 10112
RPT = N_PAD // NS

_mesh = plsc.VectorSubcoreMesh(core_axis_name="c", subcore_axis_name="s")


@functools.partial(
    pl.kernel,
    mesh=_mesh,
    out_type=[
        jax.ShapeDtypeStruct((N_PAD,), jnp.float32),
        jax.ShapeDtypeStruct((N_PAD,), jnp.float32),
    ],
    scratch_types=[
        pltpu.VMEM((CPW, CH), jnp.int32),
        pltpu.VMEM((CH,), jnp.float32),
        pltpu.VMEM((N_PAD // 8,), jnp.float32),
        pltpu.VMEM_SHARED((N_PAD,), jnp.float32),
        pltpu.SemaphoreType.DMA,
    ],
)
def _deg_call(dst_hbm, out0, out1, idx_v, ones_v, buf_v, acc_sh, sem):
    c = lax.axis_index("c")
    s = lax.axis_index("s")
    w = s * NC + c
    for i in range(CH // 16):
        ones_v[pl.ds(i * 16, 16)] = jnp.ones((16,), jnp.float32)
    for i in range(N_PAD // 8 // 16):
        buf_v[pl.ds(i * 16, 16)] = jnp.zeros((16,), jnp.float32)
    @pl.when(s < 8)
    def _():
        pltpu.sync_copy(buf_v, acc_sh.at[pl.ds(s * 1264, 1264)])

    pltpu.sync_copy(dst_hbm.at[pl.ds(w * CPW, CPW)], idx_v)
    plsc.subcore_barrier()

    def body(j, carry):
        pltpu.async_copy(ones_v, acc_sh.at[idx_v.at[j]], sem, add=True)
        return carry

    lax.fori_loop(0, CPW, body, 0)

    def drain(j, carry):
        pltpu.make_async_copy(ones_v, acc_sh.at[idx_v.at[0]], sem).wait()
        return carry

    lax.fori_loop(0, CPW, drain, 0)
    plsc.subcore_barrier()

    @pl.when(s < 8)
    def _():
        pltpu.sync_copy(acc_sh.at[pl.ds(s * 1264, 1264)], buf_v)

    @pl.when(jnp.logical_and(c == 0, s < 8))
    def _():
        pltpu.sync_copy(buf_v, out0.at[pl.ds(s * 1264, 1264)])

    @pl.when(jnp.logical_and(c == 1, s < 8))
    def _():
        pltpu.sync_copy(buf_v, out1.at[pl.ds(s * 1264, 1264)])


@functools.partial(
    pl.kernel,
    mesh=_mesh,
    out_type=[
        jax.ShapeDtypeStruct((N_PAD, D), jnp.float32),
        jax.ShapeDtypeStruct((N_PAD, D), jnp.float32),
    ],
    scratch_types=[
        pltpu.VMEM((PCH, CH), jnp.int32),
        pltpu.VMEM((PCH, CH), jnp.int32),
        pltpu.VMEM((CH, D), jnp.float32),
        pltpu.VMEM((CH, D), jnp.float32),
        pltpu.VMEM_SHARED((N_PAD, D), jnp.float32),
        pltpu.SemaphoreType.DMA,
        pltpu.SemaphoreType.DMA,
        pltpu.SemaphoreType.DMA,
        pltpu.SemaphoreType.DMA,
    ],
)
def _agg_call(hs_hbm, src_hbm, dst_hbm, out0, out1,
              sidx_v, didx_v, rows_a, rows_b, acc_sh,
              gsem_a, gsem_b, ssem_a, ssem_b):
    c = lax.axis_index("c")
    s = lax.axis_index("s")
    w = s * NC + c
    r0 = s * RPT
    @pl.when(c == 0)
    def _():
        pltpu.sync_copy(hs_hbm.at[pl.ds(r0, RPT)], acc_sh.at[pl.ds(r0, RPT)])

    @pl.when(c == 1)
    def _():
        for r in range(CH):
            for i in range(D // 16):
                rows_a[r, pl.ds(i * 16, 16)] = jnp.zeros((16,), jnp.float32)
        for k in range(RPT // CH):
            pltpu.sync_copy(rows_a,
                            acc_sh.at[pl.ds(r0 + k * CH, CH)])
        pltpu.sync_copy(rows_a.at[pl.ds(0, RPT % CH)],
                        acc_sh.at[pl.ds(r0 + RPT - RPT % CH, RPT % CH)])

    plsc.subcore_barrier()

    def _g_start(j, buf, sem):
        pltpu.async_copy(hs_hbm.at[sidx_v.at[j]], buf, sem)

    def _g_wait(buf, sem):
        pltpu.make_async_copy(hs_hbm.at[sidx_v.at[0]], buf, sem).wait()

    def _s_start(j, buf, sem):
        pltpu.async_copy(buf, acc_sh.at[didx_v.at[j]], sem, add=True)

    def _s_wait(buf, sem):
        pltpu.make_async_copy(buf, acc_sh.at[didx_v.at[0]], sem).wait()

    for p in range(CPW // PCH):
        base = w * CPW + p * PCH
        pltpu.sync_copy(src_hbm.at[pl.ds(base, PCH)], sidx_v)
        pltpu.sync_copy(dst_hbm.at[pl.ds(base, PCH)], didx_v)
        _g_start(0, rows_a, gsem_a)
        _g_start(1, rows_b, gsem_b)

        def body(t, carry):
            j = 2 * t
            _g_wait(rows_a, gsem_a)
            _s_start(j, rows_a, ssem_a)
            _g_wait(rows_b, gsem_b)
            _s_start(j + 1, rows_b, ssem_b)
            _s_wait(rows_a, ssem_a)
            _g_start(j + 2, rows_a, gsem_a)
            _s_wait(rows_b, ssem_b)
            _g_start(j + 3, rows_b, gsem_b)
            return carry

        lax.fori_loop(0, PCH // 2 - 1, body, 0)
        _g_wait(rows_a, gsem_a)
        _s_start(PCH - 2, rows_a, ssem_a)
        _g_wait(rows_b, gsem_b)
        _s_start(PCH - 1, rows_b, ssem_b)
        _s_wait(rows_a, ssem_a)
        _s_wait(rows_b, ssem_b)

    plsc.subcore_barrier()

    @pl.when(c == 0)
    def _():
        pltpu.sync_copy(acc_sh.at[pl.ds(r0, RPT)], out0.at[pl.ds(r0, RPT)])

    @pl.when(c == 1)
    def _():
        pltpu.sync_copy(acc_sh.at[pl.ds(r0, RPT)], out1.at[pl.ds(r0, RPT)])


def _dense1_body(cnt0, cnt1, x, w1, dis_o, hs_o):
    deg = cnt0[...] + cnt1[...] + 1.0
    dis = lax.rsqrt(deg)
    dis_o[...] = dis
    hs = jnp.dot(x[...], w1[...],
                 preferred_element_type=jnp.float32) * dis[:N]
    hs_o[...] = jnp.concatenate(
        [hs, jnp.zeros((N_PAD - N, D), jnp.float32)], axis=0)


def _dense1(cnt0, cnt1, x, w1):
    return pl.pallas_call(
        _dense1_body,
        out_shape=[
            jax.ShapeDtypeStruct((N_PAD, 1), jnp.float32),
            jax.ShapeDtypeStruct((N_PAD, D), jnp.float32),
        ],
    )(cnt0, cnt1, x, w1)


def _dense2_body(a0, a1, dis, b1, w2, hs2_o):
    h = jnp.maximum(dis[...] * (a0[...] + a1[...]) + b1[...], 0.0)
    hs2_o[...] = jnp.dot(h, w2[...],
                         preferred_element_type=jnp.float32) * dis[...]


def _dense2(a0, a1, dis, b1, w2):
    return pl.pallas_call(
        _dense2_body,
        out_shape=jax.ShapeDtypeStruct((N_PAD, D), jnp.float32),
    )(a0, a1, dis, b1, w2)


def _final_body(a0, a1, dis, b2, batch, wfc, bfc, out_o):
    h = jnp.maximum(dis[...] * (a0[...] + a1[...]) + b2[...], 0.0)
    gids = lax.broadcasted_iota(jnp.int32, (G, N_PAD), 0)
    onehot = jnp.where(batch[...] == gids, 1.0, 0.0)
    sums = jnp.dot(onehot, h, preferred_element_type=jnp.float32)
    counts = jnp.sum(onehot, axis=1, keepdims=True)
    pooled = sums / jnp.maximum(counts, 1.0)
    z = jnp.dot(pooled, wfc[...], preferred_element_type=jnp.float32) + bfc[...]
    out_o[...] = 1.0 / (1.0 + jnp.exp(-z))


def _final(a0, a1, dis, b2, batch, wfc, bfc):
    return pl.pallas_call(
        _final_body,
        out_shape=jax.ShapeDtypeStruct((G, 1), jnp.float32),
    )(a0, a1, dis, b2, batch, wfc, bfc)


def kernel(x, edge_index, batch, W1, b1, W2, b2, Wfc, bfc):
    src = edge_index[0]
    dst = edge_index[1]
    pad_ids = N + (jnp.arange(E_PAD - E, dtype=jnp.int32) % (N_PAD - N))
    src_p = jnp.concatenate([src, pad_ids]).reshape(E_PAD // CH, CH)
    dst_p = jnp.concatenate([dst, pad_ids]).reshape(E_PAD // CH, CH)
    batch_p = jnp.pad(batch, (0, N_PAD - N),
                      constant_values=G).reshape(1, N_PAD)

    cnt0, cnt1 = _deg_call(dst_p)
    dis, hs1 = _dense1(cnt0.reshape(N_PAD, 1), cnt1.reshape(N_PAD, 1),
                       x, W1)
    a0, a1 = _agg_call(hs1, src_p, dst_p)
    hs2 = _dense2(a0, a1, dis, b1.reshape(1, D), W2)
    a0, a1 = _agg_call(hs2, src_p, dst_p)
    return _final(a0, a1, dis, b2.reshape(1, D), batch_p, Wfc,
                  bfc.reshape(1, 1))


# speedup vs baseline: 1.0830x; 1.0830x over previous
import functools

import jax
import jax.numpy as jnp
from jax import lax
from jax.experimental import pallas as pl
from jax.experimental.pallas import tpu as pltpu
from jax.experimental.pallas import tpu_sc as plsc

N = 10000
E = 320000
D = 128
G = 64

NC = 2
NS = 16
NW = NC * NS
FPC = D // NC
CH = 128
CPW = 80
CPT = 160
PCH = 40
E_PAD = CH * CPT * NS
N_PAD = 10112
RPT = N_PAD // NS

_mesh = plsc.VectorSubcoreMesh(core_axis_name="c", subcore_axis_name="s")


@functools.partial(
    pl.kernel,
    mesh=_mesh,
    out_type=[
        jax.ShapeDtypeStruct((N_PAD,), jnp.float32),
        jax.ShapeDtypeStruct((N_PAD,), jnp.float32),
    ],
    scratch_types=[
        pltpu.VMEM((CPW, CH), jnp.int32),
        pltpu.VMEM((CH,), jnp.float32),
        pltpu.VMEM((N_PAD // 8,), jnp.float32),
        pltpu.VMEM_SHARED((N_PAD,), jnp.float32),
        pltpu.SemaphoreType.DMA,
    ],
)
def _deg_call(dst_hbm, out0, out1, idx_v, ones_v, buf_v, acc_sh, sem):
    c = lax.axis_index("c")
    s = lax.axis_index("s")
    w = s * NC + c
    for i in range(CH // 16):
        ones_v[pl.ds(i * 16, 16)] = jnp.ones((16,), jnp.float32)
    for i in range(N_PAD // 8 // 16):
        buf_v[pl.ds(i * 16, 16)] = jnp.zeros((16,), jnp.float32)
    @pl.when(s < 8)
    def _():
        pltpu.sync_copy(buf_v, acc_sh.at[pl.ds(s * 1264, 1264)])

    pltpu.sync_copy(dst_hbm.at[pl.ds(w * CPW, CPW)], idx_v)
    plsc.subcore_barrier()

    def body(j, carry):
        pltpu.async_copy(ones_v, acc_sh.at[idx_v.at[j]], sem, add=True)
        return carry

    lax.fori_loop(0, CPW, body, 0)

    def drain(j, carry):
        pltpu.make_async_copy(ones_v, acc_sh.at[idx_v.at[0]], sem).wait()
        return carry

    lax.fori_loop(0, CPW, drain, 0)
    plsc.subcore_barrier()

    @pl.when(s < 8)
    def _():
        pltpu.sync_copy(acc_sh.at[pl.ds(s * 1264, 1264)], buf_v)

    @pl.when(jnp.logical_and(c == 0, s < 8))
    def _():
        pltpu.sync_copy(buf_v, out0.at[pl.ds(s * 1264, 1264)])

    @pl.when(jnp.logical_and(c == 1, s < 8))
    def _():
        pltpu.sync_copy(buf_v, out1.at[pl.ds(s * 1264, 1264)])


@functools.partial(
    pl.kernel,
    mesh=_mesh,
    out_type=[
        jax.ShapeDtypeStruct((N_PAD, D), jnp.float32),
        jax.ShapeDtypeStruct((N_PAD, D), jnp.float32),
    ],
    scratch_types=[
        pltpu.VMEM((PCH, CH), jnp.int32),
        pltpu.VMEM((PCH, CH), jnp.int32),
        pltpu.VMEM((CH, D), jnp.float32),
        pltpu.VMEM((CH, D), jnp.float32),
        pltpu.VMEM_SHARED((N_PAD, D), jnp.float32),
        pltpu.SemaphoreType.DMA,
        pltpu.SemaphoreType.DMA,
    ],
)
def _agg_call(hs_hbm, src_hbm, dst_hbm, out0, out1,
              sidx_v, didx_v, rows_a, rows_b, acc_sh, gsem_a, gsem_b):
    c = lax.axis_index("c")
    s = lax.axis_index("s")
    w = s * NC + c
    r0 = s * RPT
    @pl.when(c == 0)
    def _():
        pltpu.sync_copy(hs_hbm.at[pl.ds(r0, RPT)], acc_sh.at[pl.ds(r0, RPT)])

    @pl.when(c == 1)
    def _():
        for r in range(CH):
            for i in range(D // 16):
                rows_a[r, pl.ds(i * 16, 16)] = jnp.zeros((16,), jnp.float32)
        for k in range(RPT // CH):
            pltpu.sync_copy(rows_a,
                            acc_sh.at[pl.ds(r0 + k * CH, CH)])
        pltpu.sync_copy(rows_a.at[pl.ds(0, RPT % CH)],
                        acc_sh.at[pl.ds(r0 + RPT - RPT % CH, RPT % CH)])

    plsc.subcore_barrier()

    def _g_start(j, buf, sem):
        pltpu.async_copy(hs_hbm.at[sidx_v.at[j]], buf, sem)

    def _g_wait(buf, sem):
        pltpu.make_async_copy(hs_hbm.at[sidx_v.at[0]], buf, sem).wait()

    def _scat(j, buf):
        pltpu.sync_copy(buf, acc_sh.at[didx_v.at[j]], add=True)

    for p in range(CPW // PCH):
        base = w * CPW + p * PCH
        pltpu.sync_copy(src_hbm.at[pl.ds(base, PCH)], sidx_v)
        pltpu.sync_copy(dst_hbm.at[pl.ds(base, PCH)], didx_v)
        _g_start(0, rows_a, gsem_a)

        def body(t, carry):
            j = 2 * t
            _g_wait(rows_a, gsem_a)
            _g_start(j + 1, rows_b, gsem_b)
            _scat(j, rows_a)
            _g_wait(rows_b, gsem_b)
            _g_start(j + 2, rows_a, gsem_a)
            _scat(j + 1, rows_b)
            return carry

        lax.fori_loop(0, PCH // 2 - 1, body, 0)
        _g_wait(rows_a, gsem_a)
        _g_start(PCH - 1, rows_b, gsem_b)
        _scat(PCH - 2, rows_a)
        _g_wait(rows_b, gsem_b)
        _scat(PCH - 1, rows_b)

    plsc.subcore_barrier()

    @pl.when(c == 0)
    def _():
        pltpu.sync_copy(acc_sh.at[pl.ds(r0, RPT)], out0.at[pl.ds(r0, RPT)])

    @pl.when(c == 1)
    def _():
        pltpu.sync_copy(acc_sh.at[pl.ds(r0, RPT)], out1.at[pl.ds(r0, RPT)])


def _dense1_body(cnt0, cnt1, x, w1, dis_o, hs_o):
    deg = cnt0[...] + cnt1[...] + 1.0
    dis = lax.rsqrt(deg)
    dis_o[...] = dis
    hs = jnp.dot(x[...], w1[...],
                 preferred_element_type=jnp.float32) * dis[:N]
    hs_o[...] = jnp.concatenate(
        [hs, jnp.zeros((N_PAD - N, D), jnp.float32)], axis=0)


def _dense1(cnt0, cnt1, x, w1):
    return pl.pallas_call(
        _dense1_body,
        out_shape=[
            jax.ShapeDtypeStruct((N_PAD, 1), jnp.float32),
            jax.ShapeDtypeStruct((N_PAD, D), jnp.float32),
        ],
    )(cnt0, cnt1, x, w1)


def _dense2_body(a0, a1, dis, b1, w2, hs2_o):
    h = jnp.maximum(dis[...] * (a0[...] + a1[...]) + b1[...], 0.0)
    hs2_o[...] = jnp.dot(h, w2[...],
                         preferred_element_type=jnp.float32) * dis[...]


def _dense2(a0, a1, dis, b1, w2):
    return pl.pallas_call(
        _dense2_body,
        out_shape=jax.ShapeDtypeStruct((N_PAD, D), jnp.float32),
    )(a0, a1, dis, b1, w2)


def _final_body(a0, a1, dis, b2, batch, wfc, bfc, out_o):
    h = jnp.maximum(dis[...] * (a0[...] + a1[...]) + b2[...], 0.0)
    gids = lax.broadcasted_iota(jnp.int32, (G, N_PAD), 0)
    onehot = jnp.where(batch[...] == gids, 1.0, 0.0)
    sums = jnp.dot(onehot, h, preferred_element_type=jnp.float32)
    counts = jnp.sum(onehot, axis=1, keepdims=True)
    pooled = sums / jnp.maximum(counts, 1.0)
    z = jnp.dot(pooled, wfc[...], preferred_element_type=jnp.float32) + bfc[...]
    out_o[...] = 1.0 / (1.0 + jnp.exp(-z))


def _final(a0, a1, dis, b2, batch, wfc, bfc):
    return pl.pallas_call(
        _final_body,
        out_shape=jax.ShapeDtypeStruct((G, 1), jnp.float32),
    )(a0, a1, dis, b2, batch, wfc, bfc)


def kernel(x, edge_index, batch, W1, b1, W2, b2, Wfc, bfc):
    src = edge_index[0]
    dst = edge_index[1]
    pad_ids = N + (jnp.arange(E_PAD - E, dtype=jnp.int32) % (N_PAD - N))
    src_p = jnp.concatenate([src, pad_ids]).reshape(E_PAD // CH, CH)
    dst_p = jnp.concatenate([dst, pad_ids]).reshape(E_PAD // CH, CH)
    batch_p = jnp.pad(batch, (0, N_PAD - N),
                      constant_values=G).reshape(1, N_PAD)

    cnt0, cnt1 = _deg_call(dst_p)
    dis, hs1 = _dense1(cnt0.reshape(N_PAD, 1), cnt1.reshape(N_PAD, 1),
                       x, W1)
    a0, a1 = _agg_call(hs1, src_p, dst_p)
    hs2 = _dense2(a0, a1, dis, b1.reshape(1, D), W2)
    a0, a1 = _agg_call(hs2, src_p, dst_p)
    return _final(a0, a1, dis, b2.reshape(1, D), batch_p, Wfc,
                  bfc.reshape(1, 1))


# speedup vs baseline: 1.1381x; 1.0509x over previous
import functools

import jax
import jax.numpy as jnp
from jax import lax
from jax.experimental import pallas as pl
from jax.experimental.pallas import tpu as pltpu
from jax.experimental.pallas import tpu_sc as plsc

N = 10000
E = 320000
D = 128
G = 64

NC = 2
NS = 16
NW = NC * NS
FPC = D // NC
CH = 128
CPW = 80
CPT = 160
PCH = 40
E_PAD = CH * CPT * NS
N_PAD = 10112
RPT = N_PAD // NS

_mesh = plsc.VectorSubcoreMesh(core_axis_name="c", subcore_axis_name="s")


@functools.partial(
    pl.kernel,
    mesh=_mesh,
    out_type=[
        jax.ShapeDtypeStruct((N_PAD,), jnp.float32),
        jax.ShapeDtypeStruct((N_PAD,), jnp.float32),
    ],
    scratch_types=[
        pltpu.VMEM((CPW, CH), jnp.int32),
        pltpu.VMEM((CH,), jnp.float32),
        pltpu.VMEM((N_PAD // 8,), jnp.float32),
        pltpu.VMEM_SHARED((N_PAD,), jnp.float32),
        pltpu.SemaphoreType.DMA,
    ],
)
def _deg_call(dst_hbm, out0, out1, idx_v, ones_v, buf_v, acc_sh, sem):
    c = lax.axis_index("c")
    s = lax.axis_index("s")
    w = s * NC + c
    for i in range(CH // 16):
        ones_v[pl.ds(i * 16, 16)] = jnp.ones((16,), jnp.float32)
    for i in range(N_PAD // 8 // 16):
        buf_v[pl.ds(i * 16, 16)] = jnp.zeros((16,), jnp.float32)
    @pl.when(s < 8)
    def _():
        pltpu.sync_copy(buf_v, acc_sh.at[pl.ds(s * 1264, 1264)])

    pltpu.sync_copy(dst_hbm.at[pl.ds(w * CPW, CPW)], idx_v)
    plsc.subcore_barrier()

    def body(j, carry):
        pltpu.async_copy(ones_v, acc_sh.at[idx_v.at[j]], sem, add=True)
        return carry

    lax.fori_loop(0, CPW, body, 0)

    def drain(j, carry):
        pltpu.make_async_copy(ones_v, acc_sh.at[idx_v.at[0]], sem).wait()
        return carry

    lax.fori_loop(0, CPW, drain, 0)
    plsc.subcore_barrier()

    @pl.when(s < 8)
    def _():
        pltpu.sync_copy(acc_sh.at[pl.ds(s * 1264, 1264)], buf_v)

    @pl.when(jnp.logical_and(c == 0, s < 8))
    def _():
        pltpu.sync_copy(buf_v, out0.at[pl.ds(s * 1264, 1264)])

    @pl.when(jnp.logical_and(c == 1, s < 8))
    def _():
        pltpu.sync_copy(buf_v, out1.at[pl.ds(s * 1264, 1264)])


@functools.partial(
    pl.kernel,
    mesh=_mesh,
    out_type=[
        jax.ShapeDtypeStruct((N_PAD, D), jnp.float32),
        jax.ShapeDtypeStruct((N_PAD, D), jnp.float32),
    ],
    scratch_types=[
        pltpu.VMEM((PCH, CH), jnp.int32),
        pltpu.VMEM((PCH, CH), jnp.int32),
        pltpu.VMEM((CH, D), jnp.float32),
        pltpu.VMEM((CH, D), jnp.float32),
        pltpu.VMEM_SHARED((N_PAD, D), jnp.float32),
        pltpu.SemaphoreType.DMA,
        pltpu.SemaphoreType.DMA,
    ],
)
def _agg_call(hs_hbm, src_hbm, dst_hbm, out0, out1,
              sidx_v, didx_v, rows_a, rows_b, acc_sh, gsem_a, gsem_b):
    c = lax.axis_index("c")
    s = lax.axis_index("s")
    w = s * NC + c
    r0 = s * RPT
    @pl.when(c == 0)
    def _():
        pltpu.sync_copy(hs_hbm.at[pl.ds(r0, RPT)], acc_sh.at[pl.ds(r0, RPT)])

    @pl.when(c == 1)
    def _():
        for r in range(CH):
            for i in range(D // 16):
                rows_a[r, pl.ds(i * 16, 16)] = jnp.zeros((16,), jnp.float32)
        for k in range(RPT // CH):
            pltpu.sync_copy(rows_a,
                            acc_sh.at[pl.ds(r0 + k * CH, CH)])
        pltpu.sync_copy(rows_a.at[pl.ds(0, RPT % CH)],
                        acc_sh.at[pl.ds(r0 + RPT - RPT % CH, RPT % CH)])

    plsc.subcore_barrier()

    def _g_start(j, buf, sem):
        pltpu.async_copy(hs_hbm.at[sidx_v.at[j]], buf, sem)

    def _g_wait(buf, sem):
        pltpu.make_async_copy(hs_hbm.at[sidx_v.at[0]], buf, sem).wait()

    def _scat(j, buf):
        pltpu.sync_copy(buf, acc_sh.at[didx_v.at[j]], add=True)

    for p in range(CPW // PCH):
        base = w * CPW + p * PCH
        pltpu.sync_copy(src_hbm.at[pl.ds(base, PCH)], sidx_v)
        pltpu.sync_copy(dst_hbm.at[pl.ds(base, PCH)], didx_v)
        _g_start(0, rows_a, gsem_a)

        def body(t, carry):
            j = 2 * t
            _g_wait(rows_a, gsem_a)
            _g_start(j + 1, rows_b, gsem_b)
            _scat(j, rows_a)
            _g_wait(rows_b, gsem_b)
            _g_start(j + 2, rows_a, gsem_a)
            _scat(j + 1, rows_b)
            return carry

        lax.fori_loop(0, PCH // 2 - 1, body, 0)
        _g_wait(rows_a, gsem_a)
        _g_start(PCH - 1, rows_b, gsem_b)
        _scat(PCH - 2, rows_a)
        _g_wait(rows_b, gsem_b)
        _scat(PCH - 1, rows_b)

    plsc.subcore_barrier()

    @pl.when(c == 0)
    def _():
        pltpu.sync_copy(acc_sh.at[pl.ds(r0, RPT)], out0.at[pl.ds(r0, RPT)])

    @pl.when(c == 1)
    def _():
        pltpu.sync_copy(acc_sh.at[pl.ds(r0, RPT)], out1.at[pl.ds(r0, RPT)])


def _dense1_body(cnt0, cnt1, x, w1, dis_o, hs_o):
    deg = cnt0[...] + cnt1[...] + 1.0
    dis = jnp.reshape(lax.rsqrt(deg), (N_PAD, 1))
    dis_o[...] = dis
    hs = jnp.dot(x[...], w1[...],
                 preferred_element_type=jnp.float32) * dis[:N]
    hs_o[...] = jnp.concatenate(
        [hs, jnp.zeros((N_PAD - N, D), jnp.float32)], axis=0)


def _dense1(cnt0, cnt1, x, w1):
    return pl.pallas_call(
        _dense1_body,
        out_shape=[
            jax.ShapeDtypeStruct((N_PAD, 1), jnp.float32),
            jax.ShapeDtypeStruct((N_PAD, D), jnp.float32),
        ],
    )(cnt0, cnt1, x, w1)


def _dense2_body(a0, a1, dis, b1, w2, hs2_o):
    h = jnp.maximum(dis[...] * (a0[...] + a1[...]) + b1[...], 0.0)
    hs2_o[...] = jnp.dot(h, w2[...],
                         preferred_element_type=jnp.float32) * dis[...]


def _dense2(a0, a1, dis, b1, w2):
    return pl.pallas_call(
        _dense2_body,
        out_shape=jax.ShapeDtypeStruct((N_PAD, D), jnp.float32),
    )(a0, a1, dis, b1, w2)


def _final_body(a0, a1, dis, b2, batch, wfc, bfc, out_o):
    h = jnp.maximum(dis[...] * (a0[...] + a1[...]) + b2[...], 0.0)
    gids = lax.broadcasted_iota(jnp.int32, (G, N_PAD), 0)
    onehot = jnp.where(batch[...] == gids, 1.0, 0.0)
    sums = jnp.dot(onehot, h, preferred_element_type=jnp.float32)
    counts = jnp.sum(onehot, axis=1, keepdims=True)
    pooled = sums / jnp.maximum(counts, 1.0)
    z = jnp.dot(pooled, wfc[...], preferred_element_type=jnp.float32) + bfc[...]
    out_o[...] = 1.0 / (1.0 + jnp.exp(-z))


def _final(a0, a1, dis, b2, batch, wfc, bfc):
    return pl.pallas_call(
        _final_body,
        out_shape=jax.ShapeDtypeStruct((G, 1), jnp.float32),
    )(a0, a1, dis, b2, batch, wfc, bfc)


def kernel(x, edge_index, batch, W1, b1, W2, b2, Wfc, bfc):
    src = edge_index[0]
    dst = edge_index[1]
    npadrows = (E_PAD - E) // CH
    flat = (lax.broadcasted_iota(jnp.int32, (npadrows, CH), 0) * CH
            + lax.broadcasted_iota(jnp.int32, (npadrows, CH), 1))
    pad2d = N + flat % (N_PAD - N)
    src_p = jnp.concatenate([src.reshape(E // CH, CH), pad2d], axis=0)
    dst_p = jnp.concatenate([dst.reshape(E // CH, CH), pad2d], axis=0)
    batch_p = jnp.pad(batch, (0, N_PAD - N),
                      constant_values=G).reshape(1, N_PAD)

    cnt0, cnt1 = _deg_call(dst_p)
    dis, hs1 = _dense1(cnt0, cnt1, x, W1)
    a0, a1 = _agg_call(hs1, src_p, dst_p)
    hs2 = _dense2(a0, a1, dis, b1.reshape(1, D), W2)
    a0, a1 = _agg_call(hs2, src_p, dst_p)
    return _final(a0, a1, dis, b2.reshape(1, D), batch_p, Wfc,
                  bfc.reshape(1, 1))
